# Initial kernel scaffold; baseline (speedup 1.0000x reference)
import functools
import jax
import jax.numpy as jnp
from jax import lax
from jax.experimental import pallas as pl
from jax.experimental.pallas import tpu as pltpu
from jax.experimental.pallas import tpu_sc as plsc

B, S, D_IN, D_OUT = 1, 2048, 768, 768
D_QUERY, N_HEADS = 256, 8
N_EXPERTS, N_SQRT, TOPK = 16384, 128, 10
D_QH = D_QUERY // 2
HK = N_HEADS * TOPK

TS = 256
NEG = -1e30


def _topk_iter(s, iota, k):
    vals, idxs = [], []
    n = s.shape[1]
    for _ in range(k):
        m = jnp.max(s, axis=1)
        eq = s >= m[:, None]
        i = jnp.min(jnp.where(eq, iota, n), axis=1)
        vals.append(m)
        idxs.append(i)
        s = jnp.where(iota == i[:, None], NEG, s)
    return vals, idxs


def _routing_body(x_ref, wq_ref, bq_ref, k1_ref, k2_ref, idx_ref, wt_ref):
    x = x_ref[...]
    q = jax.lax.dot_general(x, wq_ref[...], (((1,), (1,)), ((), ())),
                            preferred_element_type=jnp.float32)
    q = q + bq_ref[...]
    k1 = k1_ref[...]
    k2 = k2_ref[...]
    iota_n = lax.broadcasted_iota(jnp.int32, (TS, N_SQRT), 1)
    iota_c = lax.broadcasted_iota(jnp.int32, (TS, TOPK * TOPK), 1)
    idx_cols = []
    wt_cols = []
    for h in range(N_HEADS):
        q1 = q[:, h * D_QUERY: h * D_QUERY + D_QH]
        q2 = q[:, h * D_QUERY + D_QH: (h + 1) * D_QUERY]
        s1 = jax.lax.dot_general(q1, k1, (((1,), (1,)), ((), ())),
                                 preferred_element_type=jnp.float32)
        s2 = jax.lax.dot_general(q2, k2, (((1,), (1,)), ((), ())),
                                 preferred_element_type=jnp.float32)
        t1v, t1i = _topk_iter(s1, iota_n, TOPK)
        t2v, t2i = _topk_iter(s2, iota_n, TOPK)
        sc_cols = []
        ix_cols = []
        for a in range(TOPK):
            for b in range(TOPK):
                sc_cols.append((t1v[a] + t2v[b])[:, None])
                ix_cols.append((t1i[a] * N_SQRT + t2i[b])[:, None])
        all_s = jnp.concatenate(sc_cols, axis=1)
        all_i = jnp.concatenate(ix_cols, axis=1)
        fv, fj = _topk_iter(all_s, iota_c, TOPK)
        sel_e = []
        for k in range(TOPK):
            mask = iota_c == fj[k][:, None]
            sel_e.append(jnp.sum(jnp.where(mask, all_i, 0), axis=1)[:, None])
        vmax = fv[0]
        exps = [jnp.exp(v - vmax) for v in fv]
        denom = exps[0]
        for e in exps[1:]:
            denom = denom + e
        inv = 1.0 / denom
        for k in range(TOPK):
            idx_cols.append(sel_e[k])
            wt_cols.append((exps[k] * inv)[:, None])
    idx_ref[...] = jnp.concatenate(idx_cols, axis=1)
    wt_ref[...] = jnp.concatenate(wt_cols, axis=1)


def _routing(x2d, Wq, bq2d, K1, K2):
    grid = (S // TS,)
    return pl.pallas_call(
        _routing_body,
        grid=grid,
        in_specs=[
            pl.BlockSpec((TS, D_IN), lambda i: (i, 0)),
            pl.BlockSpec((D_QUERY * N_HEADS, D_IN), lambda i: (0, 0)),
            pl.BlockSpec((1, D_QUERY * N_HEADS), lambda i: (0, 0)),
            pl.BlockSpec((N_SQRT, D_QH), lambda i: (0, 0)),
            pl.BlockSpec((N_SQRT, D_QH), lambda i: (0, 0)),
        ],
        out_specs=[
            pl.BlockSpec((TS, HK), lambda i: (i, 0)),
            pl.BlockSpec((TS, HK), lambda i: (i, 0)),
        ],
        out_shape=[
            jax.ShapeDtypeStruct((S, HK), jnp.int32),
            jax.ShapeDtypeStruct((S, HK), jnp.float32),
        ],
    )(x2d, Wq, bq2d, K1, K2)


NC, NS, L = 2, 16, 16
NW = NC * NS
TPW = S // NW
NCH = D_IN // L
NJ = HK // L


def _erf(v):
    a1, a2, a3, a4, a5 = (0.254829592, -0.284496736, 1.421413741,
                          -1.453152027, 1.061405429)
    p = 0.3275911
    sgn = jnp.where(v < 0.0, -1.0, 1.0)
    av = jnp.abs(v)
    t = 1.0 / (1.0 + p * av)
    poly = ((((a5 * t + a4) * t + a3) * t + a2) * t + a1) * t
    y = 1.0 - poly * jnp.exp(-av * av)
    return sgn * y


def _expert_body(x_hbm, idx_hbm, alpha_hbm, wd_hbm, wu_hbm, out_hbm,
                 idx_v, alpha_v, x_v, rows_v, c_v, out_v, sem, sem2):
    wid = lax.axis_index("s") * NC + lax.axis_index("c")
    base = wid * TPW

    pltpu.sync_copy(idx_hbm.at[pl.ds(base, TPW)], idx_v)
    pltpu.sync_copy(alpha_hbm.at[pl.ds(base, TPW)], alpha_v)
    pltpu.sync_copy(x_hbm.at[pl.ds(base, TPW)], x_v)

    jbase = lax.iota(jnp.int32, 16) * D_IN

    def token_body(t, carry):
        pltpu.async_copy(wd_hbm.at[idx_v.at[t]], rows_v, sem).wait()
        rows_flat = rows_v.reshape(HK * D_IN)
        for jg in range(NJ):
            h = jnp.zeros((L,), jnp.float32)

            def chunk_body(cc, hacc):
                g = plsc.load_gather(rows_flat, [jbase + (jg * L * D_IN + cc)])
                return hacc + x_v[t, cc] * g

            h = lax.fori_loop(0, D_IN, chunk_body, h, unroll=8)
            a = alpha_v[t, pl.ds(jg * L, L)]
            c = a * 0.5 * h * (1.0 + _erf(h * 0.7071067811865476))
            c_v[pl.ds(jg * L, L)] = c
        pltpu.async_copy(wu_hbm.at[idx_v.at[t]], rows_v, sem).wait()
        SB = 8
        for sb in range(NCH // SB):
            acc = [jnp.zeros((L,), jnp.float32) for _ in range(SB)]

            def j_body(j, accs):
                cj = c_v[j]
                return tuple(
                    accs[u] + cj * rows_v[j, pl.ds((sb * SB + u) * L, L)]
                    for u in range(SB))

            acc = lax.fori_loop(0, HK, j_body, tuple(acc))
            for u in range(SB):
                out_v[pl.ds((sb * SB + u) * L, L)] = acc[u]
        pltpu.async_copy(out_v, out_hbm.at[base + t], sem2).wait()
        return carry

    lax.fori_loop(0, TPW, token_body, 0)


def _expert(x2d, idx, alpha, W_down, W_up):
    mesh = plsc.VectorSubcoreMesh(core_axis_name="c", subcore_axis_name="s")
    f = pl.kernel(
        _expert_body,
        out_type=jax.ShapeDtypeStruct((S, D_OUT), jnp.float32),
        mesh=mesh,
        scratch_types=[
            pltpu.VMEM((TPW, HK), jnp.int32),
            pltpu.VMEM((TPW, HK), jnp.float32),
            pltpu.VMEM((TPW, D_IN), jnp.float32),
            pltpu.VMEM((HK, D_IN), jnp.float32),
            pltpu.VMEM((HK,), jnp.float32),
            pltpu.VMEM((D_OUT,), jnp.float32),
            pltpu.SemaphoreType.DMA,
            pltpu.SemaphoreType.DMA,
        ],
    )
    return f(x2d, idx, alpha, W_down, W_up)


def kernel(x, K1, K2, Wq, bq, W_up, W_down):
    x2d = x.reshape(S, D_IN)
    bq2d = bq.reshape(1, D_QUERY * N_HEADS)
    idx, alpha = _routing(x2d, Wq, bq2d, K1, K2)
    out = _expert(x2d, idx, alpha, W_down, W_up)
    return out.reshape(B, S, D_OUT)


# speedup vs baseline: 5.5218x; 5.5218x over previous
import functools
import jax
import jax.numpy as jnp
from jax import lax
from jax.experimental import pallas as pl
from jax.experimental.pallas import tpu as pltpu
from jax.experimental.pallas import tpu_sc as plsc

B, S, D_IN, D_OUT = 1, 2048, 768, 768
D_QUERY, N_HEADS = 256, 8
N_EXPERTS, N_SQRT, TOPK = 16384, 128, 10
D_QH = D_QUERY // 2
HK = N_HEADS * TOPK

TS = 256
NEG = -1e30


def _topk_iter(s, iota, k):
    vals, idxs = [], []
    n = s.shape[1]
    for _ in range(k):
        m = jnp.max(s, axis=1)
        eq = s >= m[:, None]
        i = jnp.min(jnp.where(eq, iota, n), axis=1)
        vals.append(m)
        idxs.append(i)
        s = jnp.where(iota == i[:, None], NEG, s)
    return vals, idxs


def _routing_body(x_ref, wq_ref, bq_ref, k1_ref, k2_ref, idx_ref, wt_ref):
    x = x_ref[...]
    q = jax.lax.dot_general(x, wq_ref[...], (((1,), (1,)), ((), ())),
                            preferred_element_type=jnp.float32)
    q = q + bq_ref[...]
    k1 = k1_ref[...]
    k2 = k2_ref[...]
    iota_n = lax.broadcasted_iota(jnp.int32, (TS, N_SQRT), 1)
    iota_c = lax.broadcasted_iota(jnp.int32, (TS, TOPK * TOPK), 1)
    idx_cols = []
    wt_cols = []
    for h in range(N_HEADS):
        q1 = q[:, h * D_QUERY: h * D_QUERY + D_QH]
        q2 = q[:, h * D_QUERY + D_QH: (h + 1) * D_QUERY]
        s1 = jax.lax.dot_general(q1, k1, (((1,), (1,)), ((), ())),
                                 preferred_element_type=jnp.float32)
        s2 = jax.lax.dot_general(q2, k2, (((1,), (1,)), ((), ())),
                                 preferred_element_type=jnp.float32)
        t1v, t1i = _topk_iter(s1, iota_n, TOPK)
        t2v, t2i = _topk_iter(s2, iota_n, TOPK)
        sc_cols = []
        ix_cols = []
        for a in range(TOPK):
            for b in range(TOPK):
                sc_cols.append((t1v[a] + t2v[b])[:, None])
                ix_cols.append((t1i[a] * N_SQRT + t2i[b])[:, None])
        all_s = jnp.concatenate(sc_cols, axis=1)
        all_i = jnp.concatenate(ix_cols, axis=1)
        fv, fj = _topk_iter(all_s, iota_c, TOPK)
        sel_e = []
        for k in range(TOPK):
            mask = iota_c == fj[k][:, None]
            sel_e.append(jnp.sum(jnp.where(mask, all_i, 0), axis=1)[:, None])
        vmax = fv[0]
        exps = [jnp.exp(v - vmax) for v in fv]
        denom = exps[0]
        for e in exps[1:]:
            denom = denom + e
        inv = 1.0 / denom
        for k in range(TOPK):
            idx_cols.append(sel_e[k])
            wt_cols.append((exps[k] * inv)[:, None])
    idx_ref[...] = jnp.concatenate(idx_cols, axis=1)
    wt_ref[...] = jnp.concatenate(wt_cols, axis=1)


def _routing(x2d, Wq, bq2d, K1, K2):
    grid = (S // TS,)
    return pl.pallas_call(
        _routing_body,
        grid=grid,
        in_specs=[
            pl.BlockSpec((TS, D_IN), lambda i: (i, 0)),
            pl.BlockSpec((D_QUERY * N_HEADS, D_IN), lambda i: (0, 0)),
            pl.BlockSpec((1, D_QUERY * N_HEADS), lambda i: (0, 0)),
            pl.BlockSpec((N_SQRT, D_QH), lambda i: (0, 0)),
            pl.BlockSpec((N_SQRT, D_QH), lambda i: (0, 0)),
        ],
        out_specs=[
            pl.BlockSpec((TS, HK), lambda i: (i, 0)),
            pl.BlockSpec((TS, HK), lambda i: (i, 0)),
        ],
        out_shape=[
            jax.ShapeDtypeStruct((S, HK), jnp.int32),
            jax.ShapeDtypeStruct((S, HK), jnp.float32),
        ],
    )(x2d, Wq, bq2d, K1, K2)


NC, NS, L = 2, 16, 16
NW = NC * NS
TPW = S // NW
NCH = D_IN // L
NJ = HK // L


def _erf(v):
    a1, a2, a3, a4, a5 = (0.254829592, -0.284496736, 1.421413741,
                          -1.453152027, 1.061405429)
    p = 0.3275911
    sgn = jnp.where(v < 0.0, -1.0, 1.0)
    av = jnp.abs(v)
    t = 1.0 / (1.0 + p * av)
    poly = ((((a5 * t + a4) * t + a3) * t + a2) * t + a1) * t
    y = 1.0 - poly * jnp.exp(-av * av)
    return sgn * y


QT = 16
NQ = TPW // QT
SB = 8
NSB = NCH // SB


def _expert_body(x_hbm, idx_hbm, alpha_hbm, wd_hbm, wu_hbm, out_hbm,
                 idx_v, alpha_v, x_v, rows_a, rows_b, c_v, out_v,
                 sem_d, sem_u, sem_x, sem_o):
    wid = lax.axis_index("s") * NC + lax.axis_index("c")
    base = wid * TPW
    iota16 = lax.iota(jnp.int32, L)

    def quarter(q, qcarry):
        qbase = base + q * QT
        pltpu.sync_copy(idx_hbm.at[pl.ds(qbase, QT)], idx_v)
        pltpu.sync_copy(alpha_hbm.at[pl.ds(qbase, QT)], alpha_v)
        pltpu.sync_copy(x_hbm.at[qbase], x_v.at[0])
        pltpu.async_copy(wd_hbm.at[idx_v.at[0]], rows_a, sem_d)

        def tok(tl, carry):
            t_glob = qbase + tl
            xslot = lax.rem(tl, 2)
            pltpu.make_async_copy(wd_hbm.at[idx_v.at[tl]], rows_a,
                                  sem_d).wait()
            pltpu.async_copy(wu_hbm.at[idx_v.at[tl]], rows_b, sem_u)

            @pl.when(tl > 0)
            def _():
                pltpu.make_async_copy(x_hbm.at[t_glob], x_v.at[xslot],
                                      sem_x).wait()

            @pl.when(tl + 1 < QT)
            def _():
                pltpu.async_copy(x_hbm.at[t_glob + 1],
                                 x_v.at[lax.rem(tl + 1, 2)], sem_x)

            for jg in range(NJ):
                def cb_body(cb, accs):
                    xc = x_v[xslot, pl.ds(cb * L, L)]
                    return tuple(
                        accs[u] + xc * rows_a[jg * L + u, pl.ds(cb * L, L)]
                        for u in range(L))

                accs = lax.fori_loop(
                    0, NCH, cb_body,
                    tuple(jnp.zeros((L,), jnp.float32) for _ in range(L)),
                    unroll=1)
                h = jnp.zeros((L,), jnp.float32)
                for u in range(L):
                    h = jnp.where(iota16 == u, jnp.sum(accs[u]), h)
                a = alpha_v[tl, pl.ds(jg * L, L)]
                c = a * 0.5 * h * (1.0 + _erf(h * 0.7071067811865476))
                c_v[pl.ds(jg * L, L)] = c

            @pl.when(tl + 1 < QT)
            def _():
                pltpu.async_copy(wd_hbm.at[idx_v.at[tl + 1]], rows_a, sem_d)

            pltpu.make_async_copy(wu_hbm.at[idx_v.at[tl]], rows_b,
                                  sem_u).wait()

            for sb in range(NSB):
                def j_body(j, accs):
                    cj = c_v[pl.ds(j, L)][0]
                    return tuple(
                        accs[w] + cj * rows_b[j, pl.ds((sb * SB + w) * L, L)]
                        for w in range(SB))

                accs = lax.fori_loop(
                    0, HK, j_body,
                    tuple(jnp.zeros((L,), jnp.float32) for _ in range(SB)),
                    unroll=1)
                for w in range(SB):
                    out_v[pl.ds((sb * SB + w) * L, L)] = accs[w]
            pltpu.async_copy(out_v, out_hbm.at[t_glob], sem_o).wait()
            return carry

        lax.fori_loop(0, QT, tok, 0)
        return qcarry

    lax.fori_loop(0, NQ, quarter, 0)


def _expert(x2d, idx, alpha, W_down, W_up):
    mesh = plsc.VectorSubcoreMesh(core_axis_name="c", subcore_axis_name="s",
                                  num_cores=NC, num_subcores=NS)
    f = pl.kernel(
        _expert_body,
        out_type=jax.ShapeDtypeStruct((S, D_OUT), jnp.float32),
        mesh=mesh,
        compiler_params=pltpu.CompilerParams(needs_layout_passes=False),
        scratch_types=[
            pltpu.VMEM((QT, HK), jnp.int32),
            pltpu.VMEM((QT, HK), jnp.float32),
            pltpu.VMEM((2, D_IN), jnp.float32),
            pltpu.VMEM((HK, D_IN), jnp.float32),
            pltpu.VMEM((HK, D_IN), jnp.float32),
            pltpu.VMEM((HK + L, ), jnp.float32),
            pltpu.VMEM((D_OUT,), jnp.float32),
            pltpu.SemaphoreType.DMA,
            pltpu.SemaphoreType.DMA,
            pltpu.SemaphoreType.DMA,
            pltpu.SemaphoreType.DMA,
        ],
    )
    return f(x2d, idx, alpha, W_down, W_up)


def kernel(x, K1, K2, Wq, bq, W_up, W_down):
    x2d = x.reshape(S, D_IN)
    bq2d = bq.reshape(1, D_QUERY * N_HEADS)
    idx, alpha = _routing(x2d, Wq, bq2d, K1, K2)
    out = _expert(x2d, idx, alpha, W_down, W_up)
    return out.reshape(B, S, D_OUT)
